# baseline (speedup 1.0000x reference)
import jax
import jax.numpy as jnp
from jax.experimental import pallas as pl

ROWS_PER_BLOCK = 256


def _fused_body(x_ref, t_ref, o_ref):
    xb = x_ref[...]
    m = jnp.max(xb, axis=1, keepdims=True)
    cols = jax.lax.broadcasted_iota(jnp.int32, xb.shape, 1)
    masked = jnp.where(xb == m, cols, xb.shape[1])
    idx = jnp.min(masked, axis=1, keepdims=True)
    onehot = (cols == idx).astype(jnp.float32)
    o_ref[...] = jnp.dot(onehot, t_ref[...], preferred_element_type=jnp.float32)


def kernel(x, table):
    n, c = x.shape
    d = table.shape[1]
    return pl.pallas_call(
        _fused_body,
        grid=(n // ROWS_PER_BLOCK,),
        in_specs=[
            pl.BlockSpec((ROWS_PER_BLOCK, c), lambda i: (i, 0)),
            pl.BlockSpec((c, d), lambda i: (0, 0)),
        ],
        out_specs=pl.BlockSpec((ROWS_PER_BLOCK, d), lambda i: (i, 0)),
        out_shape=jax.ShapeDtypeStruct((n, d), jnp.float32),
    )(x, table)


# speedup vs baseline: 1.3042x; 1.3042x over previous
import jax
import jax.numpy as jnp
from jax.experimental import pallas as pl

ROWS_PER_BLOCK = 1024


def _fused_body(x_ref, t_ref, o_ref):
    xb = x_ref[...]
    m = jnp.max(xb, axis=1, keepdims=True)
    cols = jax.lax.broadcasted_iota(jnp.int32, xb.shape, 1)
    masked = jnp.where(xb == m, cols, xb.shape[1])
    idx = jnp.min(masked, axis=1, keepdims=True)
    onehot = (cols == idx).astype(jnp.float32)
    o_ref[...] = jnp.dot(onehot, t_ref[...], preferred_element_type=jnp.float32)


def kernel(x, table):
    n, c = x.shape
    d = table.shape[1]
    return pl.pallas_call(
        _fused_body,
        grid=(n // ROWS_PER_BLOCK,),
        in_specs=[
            pl.BlockSpec((ROWS_PER_BLOCK, c), lambda i: (i, 0)),
            pl.BlockSpec((c, d), lambda i: (0, 0)),
        ],
        out_specs=pl.BlockSpec((ROWS_PER_BLOCK, d), lambda i: (i, 0)),
        out_shape=jax.ShapeDtypeStruct((n, d), jnp.float32),
    )(x, table)


# speedup vs baseline: 1.3661x; 1.0475x over previous
import jax
import jax.numpy as jnp
from jax.experimental import pallas as pl

ROWS_PER_BLOCK = 2048


def _fused_body(x_ref, t_ref, o_ref):
    xb = x_ref[...]
    m = jnp.max(xb, axis=1, keepdims=True)
    cols = jax.lax.broadcasted_iota(jnp.int32, xb.shape, 1)
    masked = jnp.where(xb == m, cols, xb.shape[1])
    idx = jnp.min(masked, axis=1, keepdims=True)
    onehot = (cols == idx).astype(jnp.float32)
    o_ref[...] = jnp.dot(onehot, t_ref[...], preferred_element_type=jnp.float32)


def kernel(x, table):
    n, c = x.shape
    d = table.shape[1]
    return pl.pallas_call(
        _fused_body,
        grid=(n // ROWS_PER_BLOCK,),
        in_specs=[
            pl.BlockSpec((ROWS_PER_BLOCK, c), lambda i: (i, 0)),
            pl.BlockSpec((c, d), lambda i: (0, 0)),
        ],
        out_specs=pl.BlockSpec((ROWS_PER_BLOCK, d), lambda i: (i, 0)),
        out_shape=jax.ShapeDtypeStruct((n, d), jnp.float32),
    )(x, table)


# speedup vs baseline: 1.3676x; 1.0011x over previous
import jax
import jax.numpy as jnp
from jax.experimental import pallas as pl

ROWS_PER_BLOCK = 4096


def _fused_body(x_ref, t_ref, o_ref):
    xb = x_ref[...]
    m = jnp.max(xb, axis=1, keepdims=True)
    cols = jax.lax.broadcasted_iota(jnp.int32, xb.shape, 1)
    masked = jnp.where(xb == m, cols, xb.shape[1])
    idx = jnp.min(masked, axis=1, keepdims=True)
    onehot = (cols == idx).astype(jnp.float32)
    o_ref[...] = jnp.dot(onehot, t_ref[...], preferred_element_type=jnp.float32)


def kernel(x, table):
    n, c = x.shape
    d = table.shape[1]
    return pl.pallas_call(
        _fused_body,
        grid=(n // ROWS_PER_BLOCK,),
        in_specs=[
            pl.BlockSpec((ROWS_PER_BLOCK, c), lambda i: (i, 0)),
            pl.BlockSpec((c, d), lambda i: (0, 0)),
        ],
        out_specs=pl.BlockSpec((ROWS_PER_BLOCK, d), lambda i: (i, 0)),
        out_shape=jax.ShapeDtypeStruct((n, d), jnp.float32),
    )(x, table)
